# Initial kernel scaffold; baseline (speedup 1.0000x reference)
import jax
import jax.numpy as jnp
from jax.experimental import pallas as pl


def kernel(input):
    raise NotImplementedError("write your pallas kernel here")


# speedup vs baseline: 1.3700x; 1.3700x over previous
import jax
import jax.numpy as jnp
from jax.experimental import pallas as pl
from jax.experimental.pallas import tpu as pltpu

_C, _H, _W = 3, 4096, 4096
_BH = 256
_G = _H // _BH


def _tv_body(x_ref, nxt_ref, o_ref):
    i = pl.program_id(0)
    x = x_ref[...]

    dx = x[:, :, 1:] - x[:, :, :-1]
    s = jnp.sum(dx * dx)

    lr = x[:, _BH - 1, :]
    lrdx = lr[:, 1:] - lr[:, :-1]
    s = s - jnp.where(i == _G - 1, jnp.sum(lrdx * lrdx), 0.0)

    dy = x[:, 1:, :] - x[:, :-1, :]
    s = s + jnp.sum(dy * dy)
    dyl = x[:, 1:, _W - 1] - x[:, :-1, _W - 1]
    s = s - jnp.sum(dyl * dyl)

    sd = nxt_ref[:, 0, :] - lr
    sdv = sd[:, :-1]
    s = s + jnp.where(i < _G - 1, jnp.sum(sdv * sdv), 0.0)

    @pl.when(i == 0)
    def _():
        o_ref[...] = jnp.zeros_like(o_ref)

    o_ref[...] += s

    @pl.when(i == _G - 1)
    def _():
        o_ref[...] = jnp.sqrt(o_ref[...])


def kernel(input):
    x = input.reshape(_C, _H, _W)
    out = pl.pallas_call(
        _tv_body,
        grid=(_G,),
        in_specs=[
            pl.BlockSpec((_C, _BH, _W), lambda i: (0, i, 0)),
            pl.BlockSpec(
                (_C, 8, _W),
                lambda i: (0, jnp.minimum((i + 1) * (_BH // 8), _H // 8 - 1), 0),
            ),
        ],
        out_specs=pl.BlockSpec((1, 1, 128), lambda i: (0, 0, 0)),
        out_shape=jax.ShapeDtypeStruct((1, 1, 128), jnp.float32),
        compiler_params=pltpu.CompilerParams(
            dimension_semantics=("arbitrary",),
        ),
        name="tv_loss",
    )(x, x)
    return out[0, 0, 0]


# speedup vs baseline: 2.0408x; 1.4896x over previous
import jax
import jax.numpy as jnp
from jax.experimental import pallas as pl
from jax.experimental.pallas import tpu as pltpu

_C, _H, _W = 3, 4096, 4096
_BH = 256
_G = _H // _BH


def _tv_body(x_ref, nxt_ref, o_ref):
    i = pl.program_id(0)

    d = pltpu.roll(x_ref[...], _W - 1, 2) - x_ref[...]
    s = jnp.sum(d * d)

    e = pltpu.roll(x_ref[...], _BH - 1, 1) - x_ref[...]
    s = s + jnp.sum(e * e)

    c0 = x_ref[:, :, 0]
    cL = x_ref[:, :, _W - 1]
    w = c0 - cL
    s = s - jnp.sum(w * w)
    eL = pltpu.roll(cL, _BH - 1, 1) - cL
    s = s - jnp.sum(eL * eL)
    r0 = x_ref[:, 0, :]
    rL = x_ref[:, _BH - 1, :]
    wr = r0 - rL
    wrv = wr[:, :-1]
    s = s - jnp.sum(wrv * wrv)
    sd = nxt_ref[:, 0, :] - rL
    sdv = sd[:, :-1]
    s = s + jnp.where(i < _G - 1, jnp.sum(sdv * sdv), 0.0)
    lrdx = rL[:, 1:] - rL[:, :-1]
    s = s - jnp.where(i == _G - 1, jnp.sum(lrdx * lrdx), 0.0)

    @pl.when(i == 0)
    def _():
        o_ref[...] = jnp.zeros_like(o_ref)

    o_ref[...] += s

    @pl.when(i == _G - 1)
    def _():
        o_ref[...] = jnp.sqrt(o_ref[...])


def kernel(input):
    x = input.reshape(_C, _H, _W)
    out = pl.pallas_call(
        _tv_body,
        grid=(_G,),
        in_specs=[
            pl.BlockSpec((_C, _BH, _W), lambda i: (0, i, 0)),
            pl.BlockSpec(
                (_C, 8, _W),
                lambda i: (0, jnp.minimum((i + 1) * (_BH // 8), _H // 8 - 1), 0),
            ),
        ],
        out_specs=pl.BlockSpec((1, 1, 128), lambda i: (0, 0, 0)),
        out_shape=jax.ShapeDtypeStruct((1, 1, 128), jnp.float32),
        compiler_params=pltpu.CompilerParams(
            dimension_semantics=("arbitrary",),
        ),
        name="tv_loss",
    )(x, x)
    return out[0, 0, 0]


# speedup vs baseline: 2.0564x; 1.0076x over previous
import jax
import jax.numpy as jnp
from jax.experimental import pallas as pl
from jax.experimental.pallas import tpu as pltpu

_C, _H, _W = 3, 4096, 4096
_BH = 256
_G = _H // _BH


def _tv_body(x_ref, nxt_ref, o_ref):
    i = pl.program_id(0)

    xa = x_ref[...]
    d = pltpu.roll(xa, _W - 1, 2) - xa
    e = pltpu.roll(xa, _BH - 1, 1) - xa
    s = jnp.sum(d * d + e * e)

    c0 = x_ref[:, :, 0]
    cL = x_ref[:, :, _W - 1]
    w = c0 - cL
    s = s - jnp.sum(w * w)
    eL = pltpu.roll(cL, _BH - 1, 1) - cL
    s = s - jnp.sum(eL * eL)
    r0 = x_ref[:, 0, :]
    rL = x_ref[:, _BH - 1, :]
    wr = r0 - rL
    wrv = wr[:, :-1]
    s = s - jnp.sum(wrv * wrv)
    sd = nxt_ref[:, 0, :] - rL
    sdv = sd[:, :-1]
    s = s + jnp.where(i < _G - 1, jnp.sum(sdv * sdv), 0.0)
    lrdx = rL[:, 1:] - rL[:, :-1]
    s = s - jnp.where(i == _G - 1, jnp.sum(lrdx * lrdx), 0.0)

    @pl.when(i == 0)
    def _():
        o_ref[...] = jnp.zeros_like(o_ref)

    o_ref[...] += s

    @pl.when(i == _G - 1)
    def _():
        o_ref[...] = jnp.sqrt(o_ref[...])


def kernel(input):
    x = input.reshape(_C, _H, _W)
    out = pl.pallas_call(
        _tv_body,
        grid=(_G,),
        in_specs=[
            pl.BlockSpec((_C, _BH, _W), lambda i: (0, i, 0)),
            pl.BlockSpec(
                (_C, 8, _W),
                lambda i: (0, jnp.minimum((i + 1) * (_BH // 8), _H // 8 - 1), 0),
            ),
        ],
        out_specs=pl.BlockSpec((1, 1, 128), lambda i: (0, 0, 0)),
        out_shape=jax.ShapeDtypeStruct((1, 1, 128), jnp.float32),
        compiler_params=pltpu.CompilerParams(
            dimension_semantics=("arbitrary",),
            vmem_limit_bytes=57 * 1024 * 1024,
        ),
        name="tv_loss",
    )(x, x)
    return out[0, 0, 0]
